# baseline (speedup 1.0000x reference)
import functools

import jax
import jax.numpy as jnp
from jax import lax
from jax.experimental import pallas as pl
from jax.experimental.pallas import tpu as pltpu
from jax.experimental.pallas import tpu_sc as plsc

D = 128
N = 10000
K = 32
NPAD = 10240
NW = 32
PW = NPAD // NW
NC = 2
NS = 16
CHUNK = 64
NCHUNK = PW // CHUNK
PAD_IDX = NPAD - 1
EPS = 1e-5
_MESH = plsc.VectorSubcoreMesh(core_axis_name="c", subcore_axis_name="s")
_PREC = lax.Precision.HIGHEST


def _hist_body(idx_hbm, counts_hbm, idx_v, ones_v, zero_v, hist_sh, sem):
    c = lax.axis_index("c")
    s = lax.axis_index("s")
    wid = s * NC + c
    for i in range(8):
        ones_v[pl.ds(16 * i, 16)] = jnp.ones((16,), jnp.float32)
    for i in range(40):
        zero_v[pl.ds(16 * i, 16)] = jnp.zeros((16,), jnp.float32)
    pltpu.sync_copy(idx_hbm.at[wid], idx_v)
    pltpu.sync_copy(zero_v, hist_sh.at[pl.ds(s * 640, 640)])
    plsc.subcore_barrier()
    descs = [
        pltpu.async_copy(ones_v, hist_sh.at[idx_v.at[j]], sem, add=True)
        for j in range(80)
    ]
    for d in descs:
        d.wait()
    plsc.subcore_barrier()

    @pl.when(s == 0)
    def _():
        pltpu.sync_copy(hist_sh, counts_hbm.at[c])


@functools.partial(
    pl.kernel,
    out_type=jax.ShapeDtypeStruct((NC, NPAD), jnp.float32),
    mesh=_MESH,
    scratch_types=[
        pltpu.VMEM((80, 128), jnp.int32),
        pltpu.VMEM((128,), jnp.float32),
        pltpu.VMEM((640,), jnp.float32),
        pltpu.VMEM_SHARED((NPAD,), jnp.float32),
        pltpu.SemaphoreType.DMA,
    ],
)
def _sc_hist(idx_hbm, counts_hbm, *scratch):
    _hist_body(idx_hbm, counts_hbm, *scratch)


NGRP = 80
GN = 4


def _gather_body(rt_hbm, idx_hbm, out_hbm, idx_v, buf0, buf1, out_v,
                 sem0, sem1):
    c = lax.axis_index("c")
    s = lax.axis_index("s")
    wid = s * NC + c
    pltpu.sync_copy(idx_hbm.at[wid], idx_v)
    bufs = (buf0, buf1)
    sems = (sem0, sem1)

    def _accum(g, buf):
        for i4 in range(GN):
            for cc in range(8):
                sl = pl.ds(cc * 16, 16)
                acc = buf[K * i4, sl]
                for r in range(1, K):
                    acc = acc + buf[K * i4 + r, sl]
                out_v[GN * g + i4, sl] = acc

    pltpu.async_copy(rt_hbm.at[idx_v.at[0]], buf0, sem0)

    def _step(i, carry):
        g0 = 2 * i
        g1 = g0 + 1
        pltpu.async_copy(rt_hbm.at[idx_v.at[g1]], buf1, sem1)
        pltpu.make_async_copy(rt_hbm.at[idx_v.at[g0]], buf0, sem0).wait()
        _accum(g0, buf0)

        @pl.when(i < NGRP // 2 - 1)
        def _():
            pltpu.async_copy(rt_hbm.at[idx_v.at[g0 + 2]], buf0, sem0)

        pltpu.make_async_copy(rt_hbm.at[idx_v.at[g1]], buf1, sem1).wait()
        _accum(g1, buf1)
        return carry

    lax.fori_loop(0, NGRP // 2, _step, 0)
    pltpu.sync_copy(out_v, out_hbm.at[pl.ds(wid * PW, PW)])


@functools.partial(
    pl.kernel,
    out_type=jax.ShapeDtypeStruct((NPAD, D), jnp.float32),
    mesh=_MESH,
    scratch_types=[
        pltpu.VMEM((NGRP, 128), jnp.int32),
        pltpu.VMEM((GN * K, D), jnp.float32),
        pltpu.VMEM((GN * K, D), jnp.float32),
        pltpu.VMEM((PW, D), jnp.float32),
        pltpu.SemaphoreType.DMA,
        pltpu.SemaphoreType.DMA,
    ],
)
def _sc_gather(rt_hbm, idx_hbm, out_hbm, *scratch):
    _gather_body(rt_hbm, idx_hbm, out_hbm, *scratch)


def _tc_a_body(f_ref, w1_ref, w3_ref, g3_ref, b3_ref, gt_ref, z3_ref):
    f = f_ref[...]
    gt = lax.dot_general(f, w1_ref[...], (((0,), (1,)), ((), ())),
                         precision=_PREC,
                         preferred_element_type=jnp.float32)
    gt_ref[pl.ds(0, N), :] = gt
    gt_ref[pl.ds(N, NPAD - N), :] = jnp.zeros((NPAD - N, D), jnp.float32)
    y3 = lax.dot_general(w3_ref[...], f, (((1,), (0,)), ((), ())),
                         precision=_PREC,
                         preferred_element_type=jnp.float32)
    m = jnp.mean(y3, axis=1, keepdims=True)
    v = jnp.mean(jnp.square(y3 - m), axis=1, keepdims=True)
    yn = (y3 - m) * lax.rsqrt(v + EPS)
    z3_ref[...] = jnp.maximum(yn * g3_ref[...].T + b3_ref[...].T, 0.0)


_tc_a = pl.pallas_call(
    _tc_a_body,
    out_shape=(
        jax.ShapeDtypeStruct((NPAD, D), jnp.float32),
        jax.ShapeDtypeStruct((D, N), jnp.float32),
    ),
)


def _tc_b_body(gt_ref, counts_ref, g1_ref, b1_ref, rt_ref):
    cnt = counts_ref[...]
    countr = (cnt[0:1, :] + cnt[1:2, :])[:, :N]
    gt = gt_ref[...]
    gtr = gt[:N]
    sy = lax.dot_general(countr, gtr, (((1,), (0,)), ((), ())),
                         precision=_PREC,
                         preferred_element_type=jnp.float32)
    sy2 = lax.dot_general(countr, gtr * gtr, (((1,), (0,)), ((), ())),
                          precision=_PREC,
                          preferred_element_type=jnp.float32)
    inv = 1.0 / (N * K)
    mean = sy * inv
    var = sy2 * inv - mean * mean
    a = g1_ref[...] * lax.rsqrt(var + EPS)
    c = b1_ref[...] - mean * a
    rt_ref[...] = jnp.maximum(gt * a + c, 0.0)


_tc_b = pl.pallas_call(
    _tc_b_body,
    out_shape=jax.ShapeDtypeStruct((NPAD, D), jnp.float32),
)


def _tc_c_body(s1_ref, w2_ref, g2_ref, b2_ref, z3_ref, out_ref):
    s1 = s1_ref[...][:N]
    y2 = lax.dot_general(w2_ref[...], s1, (((1,), (1,)), ((), ())),
                         precision=_PREC,
                         preferred_element_type=jnp.float32)
    m = jnp.mean(y2, axis=1, keepdims=True)
    v = jnp.mean(jnp.square(y2 - m), axis=1, keepdims=True)
    yn = (y2 - m) * lax.rsqrt(v + EPS)
    z2 = jnp.maximum(yn * g2_ref[...].T + b2_ref[...].T, 0.0)
    out_ref[...] = z2 + z3_ref[...]


_tc_c = pl.pallas_call(
    _tc_c_body,
    out_shape=jax.ShapeDtypeStruct((D, N), jnp.float32),
)


def kernel(feature, neigh_idx, W1, g1, b1, W2, g2, b2, W3, g3, b3):
    f2d = feature.reshape(1, D, N)[0]
    idx = neigh_idx.reshape(N, K).astype(jnp.int32)
    idx_pad = jnp.concatenate(
        [idx, jnp.full((NPAD - N, K), PAD_IDX, jnp.int32)], axis=0)
    idx_node = idx_pad.reshape(NW, 80, 128)

    counts = _sc_hist(idx_node)
    gt, z3 = _tc_a(f2d, W1, W3, g3.reshape(1, D), b3.reshape(1, D))
    rt = _tc_b(gt, counts, g1.reshape(1, D), b1.reshape(1, D))
    s1t = _sc_gather(rt, idx_node)
    out2d = _tc_c(s1t, W2, g2.reshape(1, D), b2.reshape(1, D), z3)
    return out2d[None, :, :, None]


# speedup vs baseline: 2.1447x; 2.1447x over previous
import functools

import jax
import jax.numpy as jnp
from jax import lax
from jax.experimental import pallas as pl
from jax.experimental.pallas import tpu as pltpu
from jax.experimental.pallas import tpu_sc as plsc

D = 128
N = 10000
K = 32
NPAD = 10240
NW = 32
PW = NPAD // NW
NC = 2
NS = 16
CHUNK = 64
NCHUNK = PW // CHUNK
PAD_IDX = NPAD - 1
EPS = 1e-5
_MESH = plsc.VectorSubcoreMesh(core_axis_name="c", subcore_axis_name="s")
_PREC = lax.Precision.HIGHEST


def _hist_body(idx_hbm, counts_hbm, idx_v, ones_v, zero_v, hist_sh, sem):
    c = lax.axis_index("c")
    s = lax.axis_index("s")
    wid = s * NC + c
    for i in range(8):
        ones_v[pl.ds(16 * i, 16)] = jnp.ones((16,), jnp.float32)
    for i in range(40):
        zero_v[pl.ds(16 * i, 16)] = jnp.zeros((16,), jnp.float32)
    pltpu.sync_copy(idx_hbm.at[wid], idx_v)
    pltpu.sync_copy(zero_v, hist_sh.at[pl.ds(s * 640, 640)])
    plsc.subcore_barrier()
    descs = [
        pltpu.async_copy(ones_v, hist_sh.at[idx_v.at[j]], sem, add=True)
        for j in range(80)
    ]
    for d in descs:
        d.wait()
    plsc.subcore_barrier()

    @pl.when(s == 0)
    def _():
        pltpu.sync_copy(hist_sh, counts_hbm.at[c])


@functools.partial(
    pl.kernel,
    out_type=jax.ShapeDtypeStruct((NC, NPAD), jnp.float32),
    mesh=_MESH,
    scratch_types=[
        pltpu.VMEM((80, 128), jnp.int32),
        pltpu.VMEM((128,), jnp.float32),
        pltpu.VMEM((640,), jnp.float32),
        pltpu.VMEM_SHARED((NPAD,), jnp.float32),
        pltpu.SemaphoreType.DMA,
    ],
)
def _sc_hist(idx_hbm, counts_hbm, *scratch):
    _hist_body(idx_hbm, counts_hbm, *scratch)


NGRP = 80
GN = 4


def _gather_body(rt_hbm, idx_hbm, out_hbm, idx_v, rt_sh, buf0, buf1, out_v,
                 sem0, sem1):
    c = lax.axis_index("c")
    s = lax.axis_index("s")
    wid = s * NC + c
    pltpu.sync_copy(rt_hbm.at[pl.ds(s * (NPAD // NS), NPAD // NS)],
                    rt_sh.at[pl.ds(s * (NPAD // NS), NPAD // NS)])
    pltpu.sync_copy(idx_hbm.at[wid], idx_v)
    plsc.subcore_barrier()

    def _accum(g, buf):
        for i4 in range(GN):
            for cc in range(8):
                sl = pl.ds(cc * 16, 16)
                vals = [buf[K * i4 + r, sl] for r in range(K)]
                while len(vals) > 1:
                    vals = [vals[2 * j] + vals[2 * j + 1]
                            for j in range(len(vals) // 2)]
                out_v[(GN * g) % 32 + i4, sl] = vals[0]

    pltpu.async_copy(rt_sh.at[idx_v.at[0]], buf0, sem0)

    def _step(i, carry):
        g0 = 2 * i
        g1 = g0 + 1
        pltpu.async_copy(rt_sh.at[idx_v.at[g1]], buf1, sem1)
        pltpu.make_async_copy(rt_sh.at[idx_v.at[g0]], buf0, sem0).wait()
        _accum(g0, buf0)

        @pl.when(i < NGRP // 2 - 1)
        def _():
            pltpu.async_copy(rt_sh.at[idx_v.at[g0 + 2]], buf0, sem0)

        pltpu.make_async_copy(rt_sh.at[idx_v.at[g1]], buf1, sem1).wait()
        _accum(g1, buf1)

        @pl.when(i % 4 == 3)
        def _():
            pltpu.sync_copy(
                out_v, out_hbm.at[pl.ds(wid * PW + (i // 4) * 32, 32)])

        return carry

    lax.fori_loop(0, NGRP // 2, _step, 0)


@functools.partial(
    pl.kernel,
    out_type=jax.ShapeDtypeStruct((NPAD, D), jnp.float32),
    mesh=_MESH,
    scratch_types=[
        pltpu.VMEM((NGRP, 128), jnp.int32),
        pltpu.VMEM_SHARED((NPAD, D), jnp.float32),
        pltpu.VMEM((GN * K, D), jnp.float32),
        pltpu.VMEM((GN * K, D), jnp.float32),
        pltpu.VMEM((32, D), jnp.float32),
        pltpu.SemaphoreType.DMA,
        pltpu.SemaphoreType.DMA,
    ],
)
def _sc_gather(rt_hbm, idx_hbm, out_hbm, *scratch):
    _gather_body(rt_hbm, idx_hbm, out_hbm, *scratch)


def _tc_a_body(f_ref, w1_ref, w3_ref, g3_ref, b3_ref, gt_ref, z3_ref):
    f = f_ref[...]
    gt = lax.dot_general(f, w1_ref[...], (((0,), (1,)), ((), ())),
                         precision=_PREC,
                         preferred_element_type=jnp.float32)
    gt_ref[pl.ds(0, N), :] = gt
    gt_ref[pl.ds(N, NPAD - N), :] = jnp.zeros((NPAD - N, D), jnp.float32)
    y3 = lax.dot_general(w3_ref[...], f, (((1,), (0,)), ((), ())),
                         precision=_PREC,
                         preferred_element_type=jnp.float32)
    m = jnp.mean(y3, axis=1, keepdims=True)
    v = jnp.mean(jnp.square(y3 - m), axis=1, keepdims=True)
    yn = (y3 - m) * lax.rsqrt(v + EPS)
    z3_ref[...] = jnp.maximum(yn * g3_ref[...].T + b3_ref[...].T, 0.0)


_tc_a = pl.pallas_call(
    _tc_a_body,
    out_shape=(
        jax.ShapeDtypeStruct((NPAD, D), jnp.float32),
        jax.ShapeDtypeStruct((D, N), jnp.float32),
    ),
)


def _tc_b_body(gt_ref, counts_ref, g1_ref, b1_ref, rt_ref):
    cnt = counts_ref[...]
    countr = (cnt[0:1, :] + cnt[1:2, :])[:, :N]
    gt = gt_ref[...]
    gtr = gt[:N]
    sy = lax.dot_general(countr, gtr, (((1,), (0,)), ((), ())),
                         precision=_PREC,
                         preferred_element_type=jnp.float32)
    sy2 = lax.dot_general(countr, gtr * gtr, (((1,), (0,)), ((), ())),
                          precision=_PREC,
                          preferred_element_type=jnp.float32)
    inv = 1.0 / (N * K)
    mean = sy * inv
    var = sy2 * inv - mean * mean
    a = g1_ref[...] * lax.rsqrt(var + EPS)
    c = b1_ref[...] - mean * a
    rt_ref[...] = jnp.maximum(gt * a + c, 0.0)


_tc_b = pl.pallas_call(
    _tc_b_body,
    out_shape=jax.ShapeDtypeStruct((NPAD, D), jnp.float32),
)


def _tc_c_body(s1_ref, w2_ref, g2_ref, b2_ref, z3_ref, out_ref):
    s1 = s1_ref[...][:N]
    y2 = lax.dot_general(w2_ref[...], s1, (((1,), (1,)), ((), ())),
                         precision=_PREC,
                         preferred_element_type=jnp.float32)
    m = jnp.mean(y2, axis=1, keepdims=True)
    v = jnp.mean(jnp.square(y2 - m), axis=1, keepdims=True)
    yn = (y2 - m) * lax.rsqrt(v + EPS)
    z2 = jnp.maximum(yn * g2_ref[...].T + b2_ref[...].T, 0.0)
    out_ref[...] = z2 + z3_ref[...]


_tc_c = pl.pallas_call(
    _tc_c_body,
    out_shape=jax.ShapeDtypeStruct((D, N), jnp.float32),
)


def kernel(feature, neigh_idx, W1, g1, b1, W2, g2, b2, W3, g3, b3):
    f2d = feature.reshape(1, D, N)[0]
    idx = neigh_idx.reshape(N, K).astype(jnp.int32)
    idx_pad = jnp.concatenate(
        [idx, jnp.full((NPAD - N, K), PAD_IDX, jnp.int32)], axis=0)
    idx_node = idx_pad.reshape(NW, 80, 128)

    counts = _sc_hist(idx_node)
    gt, z3 = _tc_a(f2d, W1, W3, g3.reshape(1, D), b3.reshape(1, D))
    rt = _tc_b(gt, counts, g1.reshape(1, D), b1.reshape(1, D))
    s1t = _sc_gather(rt, idx_node)
    out2d = _tc_c(s1t, W2, g2.reshape(1, D), b2.reshape(1, D), z3)
    return out2d[None, :, :, None]


# speedup vs baseline: 2.5503x; 1.1891x over previous
import functools

import jax
import jax.numpy as jnp
from jax import lax
from jax.experimental import pallas as pl
from jax.experimental.pallas import tpu as pltpu
from jax.experimental.pallas import tpu_sc as plsc

D = 128
N = 10000
K = 32
NPAD = 10240
NW = 32
PW = NPAD // NW
NC = 2
NS = 16
CHUNK = 64
NCHUNK = PW // CHUNK
PAD_IDX = NPAD - 1
EPS = 1e-5
_MESH = plsc.VectorSubcoreMesh(core_axis_name="c", subcore_axis_name="s")
_PREC = lax.Precision.HIGHEST

_S1_PERM = tuple(
    32 * q + (2 * u if u < 16 else 2 * (u - 16) + 1)
    for q in range(4) for u in range(32)
)


def _hist_body(idx_hbm, counts_hbm, idx_v, ones_v, zero_v, hist_sh, sem):
    c = lax.axis_index("c")
    s = lax.axis_index("s")
    wid = s * NC + c
    for i in range(8):
        ones_v[pl.ds(16 * i, 16)] = jnp.ones((16,), jnp.float32)
    for i in range(40):
        zero_v[pl.ds(16 * i, 16)] = jnp.zeros((16,), jnp.float32)
    pltpu.sync_copy(idx_hbm.at[wid], idx_v)
    pltpu.sync_copy(zero_v, hist_sh.at[pl.ds(s * 640, 640)])
    plsc.subcore_barrier()
    descs = [
        pltpu.async_copy(ones_v, hist_sh.at[idx_v.at[j]], sem, add=True)
        for j in range(80)
    ]
    for d in descs:
        d.wait()
    plsc.subcore_barrier()

    @pl.when(s == 0)
    def _():
        pltpu.sync_copy(hist_sh, counts_hbm.at[c])


@functools.partial(
    pl.kernel,
    out_type=jax.ShapeDtypeStruct((NC, NPAD), jnp.float32),
    mesh=_MESH,
    scratch_types=[
        pltpu.VMEM((80, 128), jnp.int32),
        pltpu.VMEM((128,), jnp.float32),
        pltpu.VMEM((640,), jnp.float32),
        pltpu.VMEM_SHARED((NPAD,), jnp.float32),
        pltpu.SemaphoreType.DMA,
    ],
)
def _sc_hist(idx_hbm, counts_hbm, *scratch):
    _hist_body(idx_hbm, counts_hbm, *scratch)


NGRP = 80
GN = 4


_HI_MASK = -65536


def _gather_body(rt_hbm, idx_hbm, out_hbm, idx_v, rt_sh, buf0, buf1, out_v,
                 sem0, sem1):
    c = lax.axis_index("c")
    s = lax.axis_index("s")
    wid = s * NC + c
    pltpu.sync_copy(rt_hbm.at[pl.ds(s * (NPAD // NS), NPAD // NS)],
                    rt_sh.at[pl.ds(s * (NPAD // NS), NPAD // NS)])
    pltpu.sync_copy(idx_hbm.at[wid], idx_v)
    plsc.subcore_barrier()

    def _accum(g, buf):
        for i4 in range(GN):
            for q in range(4):
                sl = pl.ds(q * 16, 16)
                los, his = [], []
                for r in range(K):
                    xi = buf[K * i4 + r, sl]
                    los.append(lax.bitcast_convert_type(xi << 16, jnp.float32))
                    his.append(lax.bitcast_convert_type(xi & _HI_MASK, jnp.float32))
                for vals, half in ((los, 0), (his, 1)):
                    while len(vals) > 1:
                        vals = [vals[2 * j] + vals[2 * j + 1]
                                for j in range(len(vals) // 2)]
                    out_v[GN * g + i4, pl.ds(q * 32 + half * 16, 16)] = vals[0]

    pltpu.async_copy(rt_sh.at[idx_v.at[0]], buf0, sem0)

    def _step(i, carry):
        g0 = 2 * i
        g1 = g0 + 1
        pltpu.async_copy(rt_sh.at[idx_v.at[g1]], buf1, sem1)
        pltpu.make_async_copy(rt_sh.at[idx_v.at[g0]], buf0, sem0).wait()
        _accum(g0, buf0)

        @pl.when(i < NGRP // 2 - 1)
        def _():
            pltpu.async_copy(rt_sh.at[idx_v.at[g0 + 2]], buf0, sem0)

        pltpu.make_async_copy(rt_sh.at[idx_v.at[g1]], buf1, sem1).wait()
        _accum(g1, buf1)
        return carry

    lax.fori_loop(0, NGRP // 2, _step, 0)
    pltpu.sync_copy(out_v, out_hbm.at[pl.ds(wid * PW, PW)])


@functools.partial(
    pl.kernel,
    out_type=jax.ShapeDtypeStruct((NPAD, D), jnp.float32),
    mesh=_MESH,
    scratch_types=[
        pltpu.VMEM((NGRP, 128), jnp.int32),
        pltpu.VMEM_SHARED((NPAD, D // 2), jnp.int32),
        pltpu.VMEM((GN * K, D // 2), jnp.int32),
        pltpu.VMEM((GN * K, D // 2), jnp.int32),
        pltpu.VMEM((PW, D), jnp.float32),
        pltpu.SemaphoreType.DMA,
        pltpu.SemaphoreType.DMA,
    ],
)
def _sc_gather(rt_hbm, idx_hbm, out_hbm, *scratch):
    _gather_body(rt_hbm, idx_hbm, out_hbm, *scratch)


def _tc_a_body(f_ref, w1_ref, w3_ref, g3_ref, b3_ref, gt_ref, z3_ref):
    f = f_ref[...]
    gt = lax.dot_general(f, w1_ref[...], (((0,), (1,)), ((), ())),
                         precision=_PREC,
                         preferred_element_type=jnp.float32)
    gt_ref[pl.ds(0, N), :] = gt
    gt_ref[pl.ds(N, NPAD - N), :] = jnp.zeros((NPAD - N, D), jnp.float32)
    y3 = lax.dot_general(w3_ref[...], f, (((1,), (0,)), ((), ())),
                         precision=_PREC,
                         preferred_element_type=jnp.float32)
    m = jnp.mean(y3, axis=1, keepdims=True)
    v = jnp.mean(jnp.square(y3 - m), axis=1, keepdims=True)
    yn = (y3 - m) * lax.rsqrt(v + EPS)
    z3_ref[...] = jnp.maximum(yn * g3_ref[...].T + b3_ref[...].T, 0.0)


_tc_a = pl.pallas_call(
    _tc_a_body,
    out_shape=(
        jax.ShapeDtypeStruct((NPAD, D), jnp.float32),
        jax.ShapeDtypeStruct((D, N), jnp.float32),
    ),
)


def _tc_b_body(gt_ref, counts_ref, g1_ref, b1_ref, rt_ref):
    cnt = counts_ref[...]
    countr = (cnt[0:1, :] + cnt[1:2, :])[:, :N]
    gt = gt_ref[...]
    gtr = gt[:N]
    sy = lax.dot_general(countr, gtr, (((1,), (0,)), ((), ())),
                         precision=_PREC,
                         preferred_element_type=jnp.float32)
    sy2 = lax.dot_general(countr, gtr * gtr, (((1,), (0,)), ((), ())),
                          precision=_PREC,
                          preferred_element_type=jnp.float32)
    inv = 1.0 / (N * K)
    mean = sy * inv
    var = sy2 * inv - mean * mean
    a = g1_ref[...] * lax.rsqrt(var + EPS)
    c = b1_ref[...] - mean * a
    rt_ref[...] = jnp.maximum(gt * a + c, 0.0).astype(jnp.bfloat16)


_tc_b = pl.pallas_call(
    _tc_b_body,
    out_shape=jax.ShapeDtypeStruct((NPAD, D), jnp.bfloat16),
)


def _tc_c_body(s1_ref, w2_ref, g2_ref, b2_ref, z3_ref, out_ref):
    s1 = s1_ref[...][:N]
    y2 = lax.dot_general(w2_ref[...], s1, (((1,), (1,)), ((), ())),
                         precision=_PREC,
                         preferred_element_type=jnp.float32)
    m = jnp.mean(y2, axis=1, keepdims=True)
    v = jnp.mean(jnp.square(y2 - m), axis=1, keepdims=True)
    yn = (y2 - m) * lax.rsqrt(v + EPS)
    z2 = jnp.maximum(yn * g2_ref[...].T + b2_ref[...].T, 0.0)
    out_ref[...] = z2 + z3_ref[...]


_tc_c = pl.pallas_call(
    _tc_c_body,
    out_shape=jax.ShapeDtypeStruct((D, N), jnp.float32),
)


def kernel(feature, neigh_idx, W1, g1, b1, W2, g2, b2, W3, g3, b3):
    f2d = feature.reshape(1, D, N)[0]
    idx = neigh_idx.reshape(N, K).astype(jnp.int32)
    idx_pad = jnp.concatenate(
        [idx, jnp.full((NPAD - N, K), PAD_IDX, jnp.int32)], axis=0)
    idx_node = idx_pad.reshape(NW, 80, 128)

    counts = _sc_hist(idx_node)
    gt, z3 = _tc_a(f2d, W1, W3, g3.reshape(1, D), b3.reshape(1, D))
    rt = _tc_b(gt, counts, g1.reshape(1, D), b1.reshape(1, D))
    rt_i32 = lax.bitcast_convert_type(
        rt.reshape(NPAD, D // 2, 2), jnp.int32)
    s1t = _sc_gather(rt_i32, idx_node)
    w2p = W2[:, jnp.array(_S1_PERM, jnp.int32)]
    out2d = _tc_c(s1t, w2p, g2.reshape(1, D), b2.reshape(1, D), z3)
    return out2d[None, :, :, None]
